# Initial kernel scaffold; baseline (speedup 1.0000x reference)
import jax
import jax.numpy as jnp
from jax.experimental import pallas as pl


def kernel(x, W0, b0, W1, b1):
    raise NotImplementedError("write your pallas kernel here")


# speedup vs baseline: 10.0399x; 10.0399x over previous
import functools

import jax
import jax.numpy as jnp
from jax.experimental import pallas as pl
from jax.experimental.pallas import tpu as pltpu

N = 4096
D = 512
K = 31
ROW_BLK = 256


def _h_kernel(x_ref, w0_ref, b0_ref, w1_ref, b1_ref, h_ref):
    xb = x_ref[...]
    h1 = jax.lax.dot_general(xb, w0_ref[...], (((1,), (1,)), ((), ())),
                             preferred_element_type=jnp.float32)
    h1 = jnp.maximum(h1 + b0_ref[...], 0.0)
    h2 = jax.lax.dot_general(h1, w1_ref[...], (((1,), (1,)), ((), ())),
                             preferred_element_type=jnp.float32)
    h2 = h2 + b1_ref[...]
    ss = jnp.sum(h2 * h2, axis=1, keepdims=True)
    norm = jnp.maximum(jnp.sqrt(ss), 1e-12)
    h_ref[...] = h2 / norm


def _sim_topk_kernel(hb_ref, hall_ref, out_ref):
    hb = hb_ref[...]
    sim = jax.lax.dot_general(hb, hall_ref[...], (((1,), (1,)), ((), ())),
                              preferred_element_type=jnp.float32)

    neg = jnp.float32(-3.0e38)

    def body(_, work):
        m = jnp.max(work, axis=1, keepdims=True)
        return jnp.where(work >= m, neg, work)

    work = jax.lax.fori_loop(0, K - 1, body, sim)
    thresh = jnp.max(work, axis=1, keepdims=True)
    out_ref[...] = jnp.where(sim >= thresh, jnp.maximum(sim, 0.0), 0.0)


@jax.jit
def kernel(x, W0, b0, W1, b1):
    b0r = b0.reshape(1, D)
    b1r = b1.reshape(1, D)
    h = pl.pallas_call(
        _h_kernel,
        grid=(N // 512,),
        in_specs=[
            pl.BlockSpec((512, D), lambda i: (i, 0)),
            pl.BlockSpec((D, D), lambda i: (0, 0)),
            pl.BlockSpec((1, D), lambda i: (0, 0)),
            pl.BlockSpec((D, D), lambda i: (0, 0)),
            pl.BlockSpec((1, D), lambda i: (0, 0)),
        ],
        out_specs=pl.BlockSpec((512, D), lambda i: (i, 0)),
        out_shape=jax.ShapeDtypeStruct((N, D), jnp.float32),
    )(x, W0, b0r, W1, b1r)

    out = pl.pallas_call(
        _sim_topk_kernel,
        grid=(N // ROW_BLK,),
        in_specs=[
            pl.BlockSpec((ROW_BLK, D), lambda i: (i, 0)),
            pl.BlockSpec((N, D), lambda i: (0, 0)),
        ],
        out_specs=pl.BlockSpec((ROW_BLK, N), lambda i: (i, 0)),
        out_shape=jax.ShapeDtypeStruct((N, N), jnp.float32),
    )(h, h)
    return out
